# Initial kernel scaffold; baseline (speedup 1.0000x reference)
import jax
import jax.numpy as jnp
from jax.experimental import pallas as pl


def kernel(x, token_table, pos_table):
    raise NotImplementedError("write your pallas kernel here")


# speedup vs baseline: 1.4248x; 1.4248x over previous
import functools

import jax
import jax.numpy as jnp
from jax import lax
from jax.experimental import pallas as pl
from jax.experimental.pallas import tpu as pltpu
from jax.experimental.pallas import tpu_sc as plsc

_NC = 2
_NS = 16
_NW = _NC * _NS
_LANES = 16

_B, _L, _E = 4096, 200, 32
_N = _B * _L
_RPW = _N // _NW
_TROWS = 100
_SEQ_PER_CHUNK = 8
_CHUNK = _SEQ_PER_CHUNK * _L
_TPC = _CHUNK // _TROWS
_NCHUNKS = _RPW // _CHUNK
_IDX_ROWS = _N // _TROWS


def _body(idx_hbm, table_hbm, pos_hbm, out_hbm, idx_v, data_v, pos_v, gsem):
    wid = lax.axis_index("s") * _NC + lax.axis_index("c")
    row0 = wid * _RPW
    irow0 = wid * (_RPW // _TROWS)

    pltpu.sync_copy(pos_hbm, pos_v)

    @pl.loop(0, _NCHUNKS)
    def _chunk(c):
        pltpu.sync_copy(idx_hbm.at[pl.ds(irow0 + c * _TPC, _TPC), :], idx_v)
        copies = [
            pltpu.async_copy(
                table_hbm.at[idx_v.at[j]],
                data_v.at[pl.ds(j * _TROWS, _TROWS), :],
                gsem,
            )
            for j in range(_TPC)
        ]
        for cp in copies:
            cp.wait()

        @pl.loop(0, _L)
        def _pos(l):
            p0 = pos_v[l, pl.ds(0, _LANES)]
            p1 = pos_v[l, pl.ds(_LANES, _LANES)]
            for s in range(_SEQ_PER_CHUNK):
                plsc.addupdate(data_v.at[s * _L + l, pl.ds(0, _LANES)], p0)
                plsc.addupdate(data_v.at[s * _L + l, pl.ds(_LANES, _LANES)], p1)

        pltpu.sync_copy(data_v, out_hbm.at[pl.ds(row0 + c * _CHUNK, _CHUNK), :])


@functools.partial(jax.jit, static_argnums=())
def _run(idx, table, pos):
    mesh = plsc.VectorSubcoreMesh(
        core_axis_name="c", subcore_axis_name="s",
        num_cores=_NC, num_subcores=_NS,
    )
    fn = pl.kernel(
        _body,
        out_type=jax.ShapeDtypeStruct((_N, _E), jnp.float32),
        mesh=mesh,
        scratch_types=[
            pltpu.VMEM((_TPC, _TROWS), jnp.int32),
            pltpu.VMEM((_CHUNK, _E), jnp.float32),
            pltpu.VMEM((_L, _E), jnp.float32),
            pltpu.SemaphoreType.DMA,
        ],
        compiler_params=pltpu.CompilerParams(use_tc_tiling_on_sc=False),
    )
    return fn(idx, table, pos)


def kernel(x, token_table, pos_table):
    idx = x.reshape(_N).astype(jnp.int32).reshape(_IDX_ROWS, _TROWS)
    out = _run(idx, token_table, pos_table)
    return out.reshape(_B, _L, _E)
